# Initial kernel scaffold; baseline (speedup 1.0000x reference)
import jax
import jax.numpy as jnp
from jax.experimental import pallas as pl


def kernel(x, edge_index, batch, W_rel, b_rel, W_root):
    raise NotImplementedError("write your pallas kernel here")


# speedup vs baseline: 17.4478x; 17.4478x over previous
import functools

import jax
import jax.numpy as jnp
from jax import lax
from jax.experimental import pallas as pl
from jax.experimental.pallas import tpu as pltpu
from jax.experimental.pallas import tpu_sc as plsc

N = 10000
D = 256
G = 64
E = 160000

NC = 2
NS = 16
NW = NC * NS
L = 16

NPAD = 10240
SLICE = NPAD // NS
EW = 5008
EPAD = EW * NW


def _phase_a_body(x_ref, wrel_ref, wroot_ref, brel_ref, yrel_ref, xpart_ref):
    x = x_ref[:, :]
    yrel_ref[:, :] = jnp.sum(x * wrel_ref[:, :], axis=1, keepdims=True)
    xpart_ref[:, :] = (
        jnp.sum(x * wroot_ref[:, :], axis=1, keepdims=True) + brel_ref[0, 0]
    )


def _phase_a(x, w_rel, w_root, b_rel):
    return pl.pallas_call(
        _phase_a_body,
        out_shape=(
            jax.ShapeDtypeStruct((N, 1), jnp.float32),
            jax.ShapeDtypeStruct((N, 1), jnp.float32),
        ),
    )(x, w_rel, w_root, b_rel.reshape(1, 1))


def _phase_b_body(y_hbm, src_hbm, dst_hbm, out_hbm,
                  src_v, dst_v, y_v, acc_v, tmp_v, red_v, accs_sh):
    c = lax.axis_index("c")
    s = lax.axis_index("s")
    wid = s * NC + c
    base = wid * EW
    pltpu.sync_copy(src_hbm.at[pl.ds(base, EW)], src_v)
    pltpu.sync_copy(dst_hbm.at[pl.ds(base, EW)], dst_v)
    pltpu.sync_copy(y_hbm, y_v)

    zeros = jnp.zeros((L,), jnp.float32)

    def zero_body(i, carry):
        acc_v[pl.ds(i * L, L)] = zeros
        return carry

    lax.fori_loop(0, NPAD // L, zero_body, 0)

    def edge_body(i, carry):
        si = src_v[pl.ds(i * L, L)]
        di = dst_v[pl.ds(i * L, L)]
        y = plsc.load_gather(y_v, [si])
        plsc.addupdate_scatter(acc_v, [di], y)
        return carry

    lax.fori_loop(0, EW // L, edge_body, 0)

    pltpu.sync_copy(acc_v, accs_sh.at[s])
    plsc.subcore_barrier()

    col = s * SLICE
    pltpu.sync_copy(accs_sh.at[0, pl.ds(col, SLICE)], red_v)

    def add_body(i, carry):
        red_v[pl.ds(i * L, L)] = red_v[pl.ds(i * L, L)] + tmp_v[pl.ds(i * L, L)]
        return carry

    for r in range(1, NS):
        pltpu.sync_copy(accs_sh.at[r, pl.ds(col, SLICE)], tmp_v)
        lax.fori_loop(0, SLICE // L, add_body, 0)

    pltpu.sync_copy(red_v, out_hbm.at[c, pl.ds(col, SLICE)])


def _phase_b(y_rel_flat, src_pad, dst_pad):
    mesh = plsc.VectorSubcoreMesh(core_axis_name="c", subcore_axis_name="s")
    run = functools.partial(
        pl.kernel,
        mesh=mesh,
        compiler_params=pltpu.CompilerParams(needs_layout_passes=False),
        out_type=jax.ShapeDtypeStruct((NC, NPAD), jnp.float32),
        scratch_types=[
            pltpu.VMEM((EW,), jnp.int32),
            pltpu.VMEM((EW,), jnp.int32),
            pltpu.VMEM((N,), jnp.float32),
            pltpu.VMEM((NPAD,), jnp.float32),
            pltpu.VMEM((SLICE,), jnp.float32),
            pltpu.VMEM((SLICE,), jnp.float32),
            pltpu.VMEM_SHARED((NS, NPAD), jnp.float32),
        ],
    )(_phase_b_body)
    return run(y_rel_flat, src_pad, dst_pad)


def _phase_c_body(p_ref, xpart_ref, batch_ref, x_ref, out_ref):
    agg = p_ref[0:N, :] + p_ref[NPAD:NPAD + N, :]
    x_conv = agg + xpart_ref[:, :]
    b = batch_ref[:, :]
    g_iota = lax.broadcasted_iota(jnp.int32, (N, 128), 1)
    mask = b == g_iota
    maskf = mask.astype(jnp.float32)
    xb = jnp.where(mask, x_conv, jnp.float32(-1e30))
    seg_max = jnp.max(xb, axis=0, keepdims=True)
    m_n = jnp.sum(maskf * seg_max, axis=1, keepdims=True)
    ex = jnp.exp(x_conv - m_n)
    denom = jnp.sum(maskf * ex, axis=0, keepdims=True)
    d_n = jnp.sum(maskf * denom, axis=1, keepdims=True)
    scores = ex / (d_n + 1e-16)
    a = maskf * scores
    gx = lax.dot_general(
        a, x_ref[:, :], (((0,), (0,)), ((), ())),
        preferred_element_type=jnp.float32,
    )
    out_ref[:, :] = gx[0:G, :]


def _phase_c(partials, x_part, batch, x):
    return pl.pallas_call(
        _phase_c_body,
        out_shape=jax.ShapeDtypeStruct((G, D), jnp.float32),
    )(partials.reshape(NC * NPAD, 1), x_part, batch.reshape(N, 1), x)


@jax.jit
def kernel(x, edge_index, batch, W_rel, b_rel, W_root):
    y_rel, x_part = _phase_a(x, W_rel, W_root, b_rel)

    src = edge_index[0]
    dst = edge_index[1]
    npad = EPAD - E
    src_pad = jnp.concatenate([src, jnp.zeros((npad,), jnp.int32)])
    dst_pad = jnp.concatenate([dst, jnp.full((npad,), NPAD - 1, jnp.int32)])

    partials = _phase_b(y_rel.reshape(N), src_pad, dst_pad)

    return _phase_c(partials, x_part, batch, x)


# speedup vs baseline: 19.2679x; 1.1043x over previous
import functools

import jax
import jax.numpy as jnp
from jax import lax
from jax.experimental import pallas as pl
from jax.experimental.pallas import tpu as pltpu
from jax.experimental.pallas import tpu_sc as plsc

N = 10000
D = 256
G = 64
E = 160000

NC = 2
NS = 16
NW = NC * NS
L = 16

NPAD = 10240
SLICE = NPAD // NS
EW = E // NW

BLK_A = 1000


def _phase_a_body(x_ref, wrel_ref, wroot_ref, brel_ref, yrel_ref, xpart_ref):
    x = x_ref[:, :]
    yrel_ref[:, :] = jnp.sum(x * wrel_ref[:, :], axis=1, keepdims=True)
    xpart_ref[:, :] = (
        jnp.sum(x * wroot_ref[:, :], axis=1, keepdims=True) + brel_ref[0, 0]
    )


def _phase_a(x, w_rel, w_root, b_rel):
    grid = N // BLK_A
    return pl.pallas_call(
        _phase_a_body,
        grid=(grid,),
        in_specs=[
            pl.BlockSpec((BLK_A, D), lambda i: (i, 0)),
            pl.BlockSpec((1, D), lambda i: (0, 0)),
            pl.BlockSpec((1, D), lambda i: (0, 0)),
            pl.BlockSpec((1, 1), lambda i: (0, 0)),
        ],
        out_specs=(
            pl.BlockSpec((BLK_A, 1), lambda i: (i, 0)),
            pl.BlockSpec((BLK_A, 1), lambda i: (i, 0)),
        ),
        out_shape=(
            jax.ShapeDtypeStruct((N, 1), jnp.float32),
            jax.ShapeDtypeStruct((N, 1), jnp.float32),
        ),
    )(x, w_rel, w_root, b_rel.reshape(1, 1))


def _phase_b_body(y_hbm, src_hbm, dst_hbm, out_hbm,
                  src_v, dst_v, y_v, acc_v, tmp_v, red_v, accs_sh):
    c = lax.axis_index("c")
    s = lax.axis_index("s")
    wid = s * NC + c
    base = wid * EW
    pltpu.sync_copy(src_hbm.at[pl.ds(base, EW)], src_v.at[pl.ds(0, EW)])
    pltpu.sync_copy(dst_hbm.at[pl.ds(base, EW)], dst_v.at[pl.ds(0, EW)])
    pltpu.sync_copy(y_hbm, y_v)

    lanes = lax.iota(jnp.int32, L)
    tail = (EW // L) * L
    tmask = lanes < (EW - tail)
    src_v[pl.ds(tail, L)] = jnp.where(tmask, src_v[pl.ds(tail, L)], 0)
    dst_v[pl.ds(tail, L)] = jnp.where(tmask, dst_v[pl.ds(tail, L)], NPAD - 1)

    zeros = jnp.zeros((L,), jnp.float32)

    def zero_body(i, carry):
        acc_v[pl.ds(i * L, L)] = zeros
        return carry

    lax.fori_loop(0, NPAD // L, zero_body, 0)

    def edge_body(i, carry):
        si = src_v[pl.ds(i * L, L)]
        di = dst_v[pl.ds(i * L, L)]
        y = plsc.load_gather(y_v, [si])
        plsc.addupdate_scatter(acc_v, [di], y)
        return carry

    lax.fori_loop(0, pl.cdiv(EW, L), edge_body, 0)

    pltpu.sync_copy(acc_v, accs_sh.at[s])
    plsc.subcore_barrier()

    col = s * SLICE
    pltpu.sync_copy(accs_sh.at[:, pl.ds(col, SLICE)], tmp_v)

    def red_body(i, carry):
        acc = tmp_v[0, pl.ds(i * L, L)]
        for r in range(1, NS):
            acc = acc + tmp_v[r, pl.ds(i * L, L)]
        red_v[pl.ds(i * L, L)] = acc
        return carry

    lax.fori_loop(0, SLICE // L, red_body, 0)

    pltpu.sync_copy(red_v, out_hbm.at[c, pl.ds(col, SLICE)])


def _phase_b(y_rel_flat, src, dst):
    mesh = plsc.VectorSubcoreMesh(core_axis_name="c", subcore_axis_name="s")
    run = functools.partial(
        pl.kernel,
        mesh=mesh,
        compiler_params=pltpu.CompilerParams(needs_layout_passes=False),
        out_type=jax.ShapeDtypeStruct((NC, NPAD), jnp.float32),
        scratch_types=[
            pltpu.VMEM((EW + L,), jnp.int32),
            pltpu.VMEM((EW + L,), jnp.int32),
            pltpu.VMEM((N,), jnp.float32),
            pltpu.VMEM((NPAD,), jnp.float32),
            pltpu.VMEM((NS, SLICE), jnp.float32),
            pltpu.VMEM((SLICE,), jnp.float32),
            pltpu.VMEM_SHARED((NS, NPAD), jnp.float32),
        ],
    )(_phase_b_body)
    return run(y_rel_flat, src, dst)


def _phase_c_body(p_ref, xpart_ref, batch_ref, x_hbm, out_ref, x_v, sem):
    cp = pltpu.make_async_copy(x_hbm, x_v, sem)
    cp.start()

    agg = p_ref[0:N, :] + p_ref[NPAD:NPAD + N, :]
    x_conv = agg + xpart_ref[:, :]
    b = batch_ref[:, :]
    g_iota = lax.broadcasted_iota(jnp.int32, (N, 128), 1)
    mask = b == g_iota
    xb = jnp.where(mask, x_conv, jnp.float32(-1e30))
    seg_max = jnp.max(xb, axis=0, keepdims=True)
    p = jnp.where(mask, jnp.exp(xb - seg_max), 0.0)
    denom = jnp.sum(p, axis=0, keepdims=True)
    a = p / (denom + 1e-16)

    cp.wait()
    gx = lax.dot_general(
        a, x_v[:, :], (((0,), (0,)), ((), ())),
        preferred_element_type=jnp.float32,
    )
    out_ref[:, :] = gx[0:G, :]


def _phase_c(partials, x_part, batch, x):
    return pl.pallas_call(
        _phase_c_body,
        in_specs=[
            pl.BlockSpec(memory_space=pltpu.VMEM),
            pl.BlockSpec(memory_space=pltpu.VMEM),
            pl.BlockSpec(memory_space=pltpu.VMEM),
            pl.BlockSpec(memory_space=pl.ANY),
        ],
        scratch_shapes=[
            pltpu.VMEM((N, D), jnp.float32),
            pltpu.SemaphoreType.DMA,
        ],
        out_shape=jax.ShapeDtypeStruct((G, D), jnp.float32),
    )(partials.reshape(NC * NPAD, 1), x_part, batch.reshape(N, 1), x)


@jax.jit
def kernel(x, edge_index, batch, W_rel, b_rel, W_root):
    y_rel, x_part = _phase_a(x, W_rel, W_root, b_rel)
    partials = _phase_b(y_rel.reshape(N), edge_index[0], edge_index[1])
    return _phase_c(partials, x_part, batch, x)


# speedup vs baseline: 19.8976x; 1.0327x over previous
import functools

import jax
import jax.numpy as jnp
from jax import lax
from jax.experimental import pallas as pl
from jax.experimental.pallas import tpu as pltpu
from jax.experimental.pallas import tpu_sc as plsc

N = 10000
D = 256
G = 64
E = 160000

NC = 2
NS = 16
NW = NC * NS
L = 16

NPAD = 10240
SLICE = NPAD // NS
EW = E // NW

BLK_A = 1000


def _phase_a_body(x_ref, wrel_ref, wroot_ref, brel_ref, yrel_ref, xpart_ref):
    x = x_ref[:, :]
    yrel_ref[:, :] = jnp.sum(x * wrel_ref[:, :], axis=1, keepdims=True)
    xpart_ref[:, :] = (
        jnp.sum(x * wroot_ref[:, :], axis=1, keepdims=True) + brel_ref[0, 0]
    )


def _phase_a(x, w_rel, w_root, b_rel):
    grid = N // BLK_A
    return pl.pallas_call(
        _phase_a_body,
        grid=(grid,),
        in_specs=[
            pl.BlockSpec((BLK_A, D), lambda i: (i, 0)),
            pl.BlockSpec((1, D), lambda i: (0, 0)),
            pl.BlockSpec((1, D), lambda i: (0, 0)),
            pl.BlockSpec((1, 1), lambda i: (0, 0)),
        ],
        out_specs=(
            pl.BlockSpec((BLK_A, 1), lambda i: (i, 0)),
            pl.BlockSpec((BLK_A, 1), lambda i: (i, 0)),
        ),
        out_shape=(
            jax.ShapeDtypeStruct((N, 1), jnp.float32),
            jax.ShapeDtypeStruct((N, 1), jnp.float32),
        ),
    )(x, w_rel, w_root, b_rel.reshape(1, 1))


def _phase_b_body(y_hbm, src_hbm, dst_hbm, out_hbm,
                  src_v, dst_v, y_v, acc_v, tmp_v, red_v, accs_sh):
    c = lax.axis_index("c")
    s = lax.axis_index("s")
    wid = s * NC + c
    base = wid * EW
    pltpu.sync_copy(src_hbm.at[pl.ds(base, EW)], src_v.at[pl.ds(0, EW)])
    pltpu.sync_copy(dst_hbm.at[pl.ds(base, EW)], dst_v.at[pl.ds(0, EW)])
    pltpu.sync_copy(y_hbm, y_v)

    lanes = lax.iota(jnp.int32, L)
    tail = (EW // L) * L
    tmask = lanes < (EW - tail)
    src_v[pl.ds(tail, L)] = jnp.where(tmask, src_v[pl.ds(tail, L)], 0)
    dst_v[pl.ds(tail, L)] = jnp.where(tmask, dst_v[pl.ds(tail, L)], NPAD - 1)

    zeros = jnp.zeros((L,), jnp.float32)

    ZU = 8

    def zero_body(i, carry):
        for u in range(ZU):
            acc_v[pl.ds(i * (ZU * L) + u * L, L)] = zeros
        return carry

    lax.fori_loop(0, NPAD // (ZU * L), zero_body, 0)

    EU = 4
    NIT = pl.cdiv(EW, L)
    NFULL = NIT // EU

    def edge_body(i, carry):
        for u in range(EU):
            off = i * (EU * L) + u * L
            si = src_v[pl.ds(off, L)]
            di = dst_v[pl.ds(off, L)]
            y = plsc.load_gather(y_v, [si])
            plsc.addupdate_scatter(acc_v, [di], y)
        return carry

    lax.fori_loop(0, NFULL, edge_body, 0)

    for k in range(NFULL * EU, NIT):
        si = src_v[pl.ds(k * L, L)]
        di = dst_v[pl.ds(k * L, L)]
        y = plsc.load_gather(y_v, [si])
        plsc.addupdate_scatter(acc_v, [di], y)

    pltpu.sync_copy(acc_v, accs_sh.at[s])
    plsc.subcore_barrier()

    col = s * SLICE
    pltpu.sync_copy(accs_sh.at[:, pl.ds(col, SLICE)], tmp_v)

    def red_body(i, carry):
        acc = tmp_v[0, pl.ds(i * L, L)]
        for r in range(1, NS):
            acc = acc + tmp_v[r, pl.ds(i * L, L)]
        red_v[pl.ds(i * L, L)] = acc
        return carry

    lax.fori_loop(0, SLICE // L, red_body, 0)

    pltpu.sync_copy(red_v, out_hbm.at[c, pl.ds(col, SLICE)])


def _phase_b(y_rel_flat, src, dst):
    mesh = plsc.VectorSubcoreMesh(core_axis_name="c", subcore_axis_name="s")
    run = functools.partial(
        pl.kernel,
        mesh=mesh,
        compiler_params=pltpu.CompilerParams(needs_layout_passes=False),
        out_type=jax.ShapeDtypeStruct((NC, NPAD), jnp.float32),
        scratch_types=[
            pltpu.VMEM((EW + L,), jnp.int32),
            pltpu.VMEM((EW + L,), jnp.int32),
            pltpu.VMEM((N,), jnp.float32),
            pltpu.VMEM((NPAD,), jnp.float32),
            pltpu.VMEM((NS, SLICE), jnp.float32),
            pltpu.VMEM((SLICE,), jnp.float32),
            pltpu.VMEM_SHARED((NS, NPAD), jnp.float32),
        ],
    )(_phase_b_body)
    return run(y_rel_flat, src, dst)


def _phase_c_body(p_ref, xpart_ref, batch_ref, x_hbm, out_ref, x_v, sem):
    cp = pltpu.make_async_copy(x_hbm, x_v, sem)
    cp.start()

    agg = p_ref[0:N, :] + p_ref[NPAD:NPAD + N, :]
    x_conv = agg + xpart_ref[:, :]
    b = batch_ref[:, :]
    g_iota = lax.broadcasted_iota(jnp.int32, (N, 128), 1)
    mask = b == g_iota
    xb = jnp.where(mask, x_conv, jnp.float32(-1e30))
    seg_max = jnp.max(xb, axis=0, keepdims=True)
    p = jnp.where(mask, jnp.exp(xb - seg_max), 0.0)
    denom = jnp.sum(p, axis=0, keepdims=True)
    a = p / (denom + 1e-16)

    cp.wait()
    gx = lax.dot_general(
        a, x_v[:, :], (((0,), (0,)), ((), ())),
        preferred_element_type=jnp.float32,
    )
    out_ref[:, :] = gx[0:G, :]


def _phase_c(partials, x_part, batch, x):
    return pl.pallas_call(
        _phase_c_body,
        in_specs=[
            pl.BlockSpec(memory_space=pltpu.VMEM),
            pl.BlockSpec(memory_space=pltpu.VMEM),
            pl.BlockSpec(memory_space=pltpu.VMEM),
            pl.BlockSpec(memory_space=pl.ANY),
        ],
        scratch_shapes=[
            pltpu.VMEM((N, D), jnp.float32),
            pltpu.SemaphoreType.DMA,
        ],
        out_shape=jax.ShapeDtypeStruct((G, D), jnp.float32),
    )(partials.reshape(NC * NPAD, 1), x_part, batch.reshape(N, 1), x)


@jax.jit
def kernel(x, edge_index, batch, W_rel, b_rel, W_root):
    y_rel, x_part = _phase_a(x, W_rel, W_root, b_rel)
    partials = _phase_b(y_rel.reshape(N), edge_index[0], edge_index[1])
    return _phase_c(partials, x_part, batch, x)


# speedup vs baseline: 20.1058x; 1.0105x over previous
import functools

import jax
import jax.numpy as jnp
from jax import lax
from jax.experimental import pallas as pl
from jax.experimental.pallas import tpu as pltpu
from jax.experimental.pallas import tpu_sc as plsc

N = 10000
D = 256
G = 64
E = 160000

NC = 2
NS = 16
NW = NC * NS
L = 16

NPAD = 10240
SLICE = NPAD // NS
EW = E // NW

BLK_A = 1000


def _phase_a_body(x_ref, wrel_ref, wroot_ref, brel_ref, yrel_ref, xpart_ref):
    x = x_ref[:, :]
    yrel_ref[:, :] = jnp.sum(x * wrel_ref[:, :], axis=1, keepdims=True)
    xpart_ref[:, :] = (
        jnp.sum(x * wroot_ref[:, :], axis=1, keepdims=True) + brel_ref[0, 0]
    )


def _phase_a(x, w_rel, w_root, b_rel):
    grid = N // BLK_A
    return pl.pallas_call(
        _phase_a_body,
        grid=(grid,),
        in_specs=[
            pl.BlockSpec((BLK_A, D), lambda i: (i, 0)),
            pl.BlockSpec((1, D), lambda i: (0, 0)),
            pl.BlockSpec((1, D), lambda i: (0, 0)),
            pl.BlockSpec((1, 1), lambda i: (0, 0)),
        ],
        out_specs=(
            pl.BlockSpec((BLK_A, 1), lambda i: (i, 0)),
            pl.BlockSpec((BLK_A, 1), lambda i: (i, 0)),
        ),
        out_shape=(
            jax.ShapeDtypeStruct((N, 1), jnp.float32),
            jax.ShapeDtypeStruct((N, 1), jnp.float32),
        ),
    )(x, w_rel, w_root, b_rel.reshape(1, 1))


def _phase_b_body(y_hbm, src_hbm, dst_hbm, out_hbm,
                  src_v, dst_v, y_v, acc_v, tmp_v, red_v, accs_sh, sem):
    c = lax.axis_index("c")
    s = lax.axis_index("s")
    wid = s * NC + c
    base = wid * EW
    cp_src = pltpu.make_async_copy(
        src_hbm.at[pl.ds(base, EW)], src_v.at[pl.ds(0, EW)], sem)
    cp_dst = pltpu.make_async_copy(
        dst_hbm.at[pl.ds(base, EW)], dst_v.at[pl.ds(0, EW)], sem)
    cp_y = pltpu.make_async_copy(y_hbm, y_v, sem)
    cp_src.start()
    cp_dst.start()
    cp_y.start()

    zeros = jnp.zeros((L,), jnp.float32)

    ZU = 8

    def zero_body(i, carry):
        for u in range(ZU):
            acc_v[pl.ds(i * (ZU * L) + u * L, L)] = zeros
        return carry

    lax.fori_loop(0, NPAD // (ZU * L), zero_body, 0)

    cp_src.wait()
    cp_dst.wait()
    cp_y.wait()

    lanes = lax.iota(jnp.int32, L)
    tail = (EW // L) * L
    tmask = lanes < (EW - tail)
    src_v[pl.ds(tail, L)] = jnp.where(tmask, src_v[pl.ds(tail, L)], 0)
    dst_v[pl.ds(tail, L)] = jnp.where(tmask, dst_v[pl.ds(tail, L)], NPAD - 1)

    EU = 4
    NIT = pl.cdiv(EW, L)
    NFULL = NIT // EU

    def edge_body(i, carry):
        for u in range(EU):
            off = i * (EU * L) + u * L
            si = src_v[pl.ds(off, L)]
            di = dst_v[pl.ds(off, L)]
            y = plsc.load_gather(y_v, [si])
            plsc.addupdate_scatter(acc_v, [di], y)
        return carry

    lax.fori_loop(0, NFULL, edge_body, 0)

    for k in range(NFULL * EU, NIT):
        si = src_v[pl.ds(k * L, L)]
        di = dst_v[pl.ds(k * L, L)]
        y = plsc.load_gather(y_v, [si])
        plsc.addupdate_scatter(acc_v, [di], y)


    pltpu.sync_copy(acc_v, accs_sh.at[s])
    plsc.subcore_barrier()

    col = s * SLICE
    pltpu.sync_copy(accs_sh.at[:, pl.ds(col, SLICE)], tmp_v)

    def red_body(i, carry):
        acc = tmp_v[0, pl.ds(i * L, L)]
        for r in range(1, NS):
            acc = acc + tmp_v[r, pl.ds(i * L, L)]
        red_v[pl.ds(i * L, L)] = acc
        return carry

    lax.fori_loop(0, SLICE // L, red_body, 0)

    pltpu.sync_copy(red_v, out_hbm.at[c, pl.ds(col, SLICE)])


def _phase_b(y_rel_flat, src, dst):
    mesh = plsc.VectorSubcoreMesh(core_axis_name="c", subcore_axis_name="s")
    run = functools.partial(
        pl.kernel,
        mesh=mesh,
        compiler_params=pltpu.CompilerParams(needs_layout_passes=False),
        out_type=jax.ShapeDtypeStruct((NC, NPAD), jnp.float32),
        scratch_types=[
            pltpu.VMEM((EW + L,), jnp.int32),
            pltpu.VMEM((EW + L,), jnp.int32),
            pltpu.VMEM((N,), jnp.float32),
            pltpu.VMEM((NPAD,), jnp.float32),
            pltpu.VMEM((NS, SLICE), jnp.float32),
            pltpu.VMEM((SLICE,), jnp.float32),
            pltpu.VMEM_SHARED((NS, NPAD), jnp.float32),
            pltpu.SemaphoreType.DMA,
        ],
    )(_phase_b_body)
    return run(y_rel_flat, src, dst)


def _phase_c_body(p_ref, xpart_ref, batch_ref, x_hbm, out_ref, x_v, sem):
    cp = pltpu.make_async_copy(x_hbm, x_v, sem)
    cp.start()

    agg = p_ref[0:N, :] + p_ref[NPAD:NPAD + N, :]
    x_conv = agg + xpart_ref[:, :]
    b = batch_ref[:, :]
    g_iota = lax.broadcasted_iota(jnp.int32, (N, 128), 1)
    mask = b == g_iota
    xb = jnp.where(mask, x_conv, jnp.float32(-1e30))
    seg_max = jnp.max(xb, axis=0, keepdims=True)
    p = jnp.where(mask, jnp.exp(xb - seg_max), 0.0)
    denom = jnp.sum(p, axis=0, keepdims=True)
    a = p / (denom + 1e-16)

    cp.wait()
    gx = lax.dot_general(
        a, x_v[:, :], (((0,), (0,)), ((), ())),
        preferred_element_type=jnp.float32,
    )
    out_ref[:, :] = gx[0:G, :]


def _phase_c(partials, x_part, batch, x):
    return pl.pallas_call(
        _phase_c_body,
        in_specs=[
            pl.BlockSpec(memory_space=pltpu.VMEM),
            pl.BlockSpec(memory_space=pltpu.VMEM),
            pl.BlockSpec(memory_space=pltpu.VMEM),
            pl.BlockSpec(memory_space=pl.ANY),
        ],
        scratch_shapes=[
            pltpu.VMEM((N, D), jnp.float32),
            pltpu.SemaphoreType.DMA,
        ],
        out_shape=jax.ShapeDtypeStruct((G, D), jnp.float32),
    )(partials.reshape(NC * NPAD, 1), x_part, batch.reshape(N, 1), x)


@jax.jit
def kernel(x, edge_index, batch, W_rel, b_rel, W_root):
    y_rel, x_part = _phase_a(x, W_rel, W_root, b_rel)
    partials = _phase_b(y_rel.reshape(N), edge_index[0], edge_index[1])
    return _phase_c(partials, x_part, batch, x)
